# baseline (speedup 1.0000x reference)
import jax
import jax.numpy as jnp
from jax.experimental import pallas as pl
from jax.experimental.pallas import tpu as pltpu

D_MODEL = 768
N_VOICES = 4


def _pe_add_kernel(ts_ref, v_ref, x_ref, o_ref, pe_ref):
    bt = ts_ref.shape[0]

    @pl.when(pl.program_id(1) == 0)
    def _build_pe():
        ts = ts_ref[...]
        t_pe = jnp.repeat(ts, N_VOICES, axis=0)
        v_pe = pltpu.repeat(v_ref[...], bt, axis=0)
        pe_ref[...] = t_pe + v_pe

    o_ref[...] = x_ref[...] + pe_ref[...][None]


def kernel(x, timestep_emb, voice_emb):
    B, L, D = x.shape
    T = L // N_VOICES
    ts = timestep_emb[:T]

    BT = 512
    BB = 2
    BL = BT * N_VOICES
    grid = (T // BT, B // BB)
    return pl.pallas_call(
        _pe_add_kernel,
        grid=grid,
        in_specs=[
            pl.BlockSpec((BT, D), lambda i, b: (i, 0)),
            pl.BlockSpec((N_VOICES, D), lambda i, b: (0, 0)),
            pl.BlockSpec((BB, BL, D), lambda i, b: (b, i, 0)),
        ],
        out_specs=pl.BlockSpec((BB, BL, D), lambda i, b: (b, i, 0)),
        out_shape=jax.ShapeDtypeStruct((B, L, D), x.dtype),
        scratch_shapes=[pltpu.VMEM((BL, D), jnp.float32)],
        compiler_params=pltpu.CompilerParams(
            vmem_limit_bytes=100 * 1024 * 1024,
        ),
    )(ts, voice_emb, x)
